# baseline (speedup 1.0000x reference)
import functools

import jax
import jax.numpy as jnp
from jax.experimental import pallas as pl


def _gate_body(x_ref, w_ref, o_ref):
    x = x_ref[...].astype(jnp.bfloat16)
    w = w_ref[...].astype(jnp.bfloat16)
    o_ref[...] = jax.lax.dot_general(
        x, w, (((1,), (1,)), ((), ())),
        preferred_element_type=jnp.float32)


@functools.partial(jax.jit, static_argnames=())
def kernel(x, W):
    tokens, hidden = x.shape
    experts = W.shape[0]
    bm = 1024
    return pl.pallas_call(
        _gate_body,
        grid=(tokens // bm,),
        in_specs=[
            pl.BlockSpec((bm, hidden), lambda i: (i, 0)),
            pl.BlockSpec((experts, hidden), lambda i: (0, 0)),
        ],
        out_specs=pl.BlockSpec((bm, experts), lambda i: (i, 0)),
        out_shape=jax.ShapeDtypeStruct((tokens, experts), jnp.float32),
    )(x, W)
